# baseline (speedup 1.0000x reference)
import functools

import jax
import jax.numpy as jnp
from jax import lax
from jax.experimental import pallas as pl
from jax.experimental.pallas import tpu as pltpu
from jax.experimental.pallas import tpu_sc as plsc

N = 10000
E = 320000
D = 128
H = 128
C = 64

NC = 2
NS = 16
NW = NC * NS

CHUNK = 128
NCHUNKS = 79
EPW = NCHUNKS * CHUNK
E_PAD = NW * EPW
NPAD = 10240
RPT = NPAD // NS
RBLK = RPT // CHUNK
NROW = NPAD // CHUNK

NCHUNKS_A = 81
EPW_A = NCHUNKS_A * CHUNK
E_PAD_A = NW * EPW_A
NBUF = 3

_MESH = plsc.VectorSubcoreMesh(core_axis_name="c", subcore_axis_name="s")


@functools.partial(
    pl.kernel,
    out_type=jax.ShapeDtypeStruct((NC, NROW, CHUNK), jnp.float32),
    mesh=_MESH,
    scratch_types=[
        pltpu.VMEM((NCHUNKS, CHUNK), jnp.int32),
        pltpu.VMEM((NROW, CHUNK), jnp.float32),
        pltpu.VMEM((NROW // NS, CHUNK), jnp.float32),
        pltpu.VMEM((NROW,), jnp.int32),
        pltpu.VMEM_SHARED((NROW, CHUNK), jnp.float32),
    ],
    compiler_params=pltpu.CompilerParams(use_tc_tiling_on_sc=False,
                                         needs_layout_passes=False),
)
def _sc_degree(col_hbm, out_hbm, colbuf, hist, bounce, idbuf, acc):
    cid = lax.axis_index("c")
    sid = lax.axis_index("s")
    wid = sid * NC + cid
    pltpu.sync_copy(col_hbm.at[wid], colbuf)

    zero16 = jnp.zeros((16,), jnp.float32)
    one16 = jnp.ones((16,), jnp.float32)
    for r in range(NROW // NS):
        for j in range(CHUNK // 16):
            bounce[r, pl.ds(j * 16, 16)] = zero16
    for k in range(NROW // 16):
        idbuf[pl.ds(k * 16, 16)] = lax.iota(jnp.int32, 16) + (k * 16)

    def zrow(r, carry):
        for j in range(CHUNK // 16):
            hist[r, pl.ds(j * 16, 16)] = zero16
        return carry

    lax.fori_loop(0, NROW, zrow, 0)
    pltpu.sync_copy(bounce, acc.at[pl.ds(sid * (NROW // NS), NROW // NS)])

    def body(c, carry):
        for j in range(CHUNK // 16):
            idx = colbuf[c, pl.ds(j * 16, 16)]
            plsc.addupdate_scatter(
                hist, [lax.shift_right_logical(idx, 7),
                       lax.bitwise_and(idx, 127)], one16)
        return carry

    lax.fori_loop(0, NCHUNKS, body, 0)
    plsc.subcore_barrier()
    pltpu.sync_copy(hist, acc.at[idbuf], add=True)
    plsc.subcore_barrier()
    pltpu.sync_copy(acc.at[pl.ds(sid * (NROW // NS), NROW // NS)], bounce)
    pltpu.sync_copy(bounce,
                    out_hbm.at[cid, pl.ds(sid * (NROW // NS), NROW // NS)])


def _make_sc_agg(width):
    chunk = CHUNK
    nchunks = NCHUNKS_A
    rblk = RPT // chunk

    @functools.partial(
        pl.kernel,
        out_type=jax.ShapeDtypeStruct((NC, NPAD, width), jnp.bfloat16),
        mesh=_MESH,
        scratch_types=(
            [pltpu.VMEM((nchunks, chunk), jnp.int32),
             pltpu.VMEM((nchunks, chunk), jnp.int32)]
            + [pltpu.VMEM((chunk, width), jnp.bfloat16)
               for _ in range(NBUF)]
            + [pltpu.VMEM_SHARED((NPAD, width), jnp.bfloat16),
               pltpu.VMEM_SHARED((NPAD, width), jnp.bfloat16)]
            + [pltpu.SemaphoreType.DMA for _ in range(2 * NBUF)]
        ),
        compiler_params=pltpu.CompilerParams(use_tc_tiling_on_sc=False),
    )
    def agg(g_hbm, row_hbm, col_hbm, out_hbm, rowbuf, colbuf,
            buf0, buf1, buf2, table, acc, sg0, sg1, sg2, ss0, ss1, ss2):
        bufs = (buf0, buf1, buf2)
        sgs = (sg0, sg1, sg2)
        sss = (ss0, ss1, ss2)
        cid = lax.axis_index("c")
        sid = lax.axis_index("s")
        wid = sid * NC + cid
        pltpu.sync_copy(row_hbm.at[wid], rowbuf)
        pltpu.sync_copy(col_hbm.at[wid], colbuf)

        for k in range(rblk):
            pltpu.sync_copy(g_hbm.at[pl.ds(sid * RPT + k * chunk, chunk)],
                            buf0)
            pltpu.sync_copy(buf0, table.at[pl.ds(sid * RPT + k * chunk,
                                                 chunk)])

        zero32 = jnp.zeros((32,), jnp.bfloat16)

        def zrow(r, carry):
            for j in range(width // 32):
                buf0[r, pl.ds(j * 32, 32)] = zero32
            return carry

        lax.fori_loop(0, chunk, zrow, 0)
        for k in range(rblk):
            pltpu.sync_copy(buf0, acc.at[pl.ds(sid * RPT + k * chunk, chunk)])
        plsc.subcore_barrier()

        for j in range(NBUF):
            pltpu.async_copy(table.at[rowbuf.at[j]], bufs[j], sgs[j])

        def body(i, carry):
            c0 = NBUF * i
            for j in range(NBUF):
                pltpu.make_async_copy(table.at[rowbuf.at[c0 + j]], bufs[j],
                                      sgs[j]).wait()
                pltpu.make_async_copy(bufs[j], acc.at[colbuf.at[c0 + j]],
                                      sss[j]).start(add=True)
            for j in range(NBUF):
                cn = c0 + j + NBUF

                @pl.when(cn < nchunks)
                def _(j=j, cn=cn):
                    pltpu.make_async_copy(bufs[j], acc.at[colbuf.at[cn]],
                                          sss[j]).wait()
                    pltpu.async_copy(table.at[rowbuf.at[cn]], bufs[j], sgs[j])

            return carry

        lax.fori_loop(0, nchunks // NBUF, body, 0)
        for j in range(NBUF):
            pltpu.make_async_copy(bufs[j], acc.at[colbuf.at[0]],
                                  sss[j]).wait()
        plsc.subcore_barrier()
        for k in range(rblk):
            pltpu.sync_copy(acc.at[pl.ds(sid * RPT + k * chunk, chunk)], buf0)
            pltpu.sync_copy(buf0, out_hbm.at[cid,
                                             pl.ds(sid * RPT + k * chunk,
                                                   chunk)])

    return agg


_sc_agg_h = _make_sc_agg(H)
_sc_agg_c = _make_sc_agg(C)


_BLK = 2048
_GRID = (N + _BLK - 1) // _BLK


def _dense1_body(x_ref, w1_ref, deg_ref, g1b_ref, dis_ref):
    deg = deg_ref[0, :] + deg_ref[1, :] + 1.0
    dis = lax.rsqrt(deg)[:, None]
    g1 = dis * jnp.dot(x_ref[...], w1_ref[...],
                       preferred_element_type=jnp.float32)
    g1b_ref[...] = g1.astype(jnp.bfloat16)
    dis_ref[...] = dis


_dense1 = pl.pallas_call(
    _dense1_body,
    grid=(_GRID,),
    in_specs=[
        pl.BlockSpec((_BLK, D), lambda i: (i, 0)),
        pl.BlockSpec((D, H), lambda i: (0, 0)),
        pl.BlockSpec((2, _BLK), lambda i: (0, i)),
    ],
    out_specs=(pl.BlockSpec((_BLK, H), lambda i: (i, 0)),
               pl.BlockSpec((_BLK, 1), lambda i: (i, 0))),
    out_shape=(jax.ShapeDtypeStruct((N, H), jnp.bfloat16),
               jax.ShapeDtypeStruct((N, 1), jnp.float32)),
)


def _dense2_body(agg_ref, dis_ref, b1_ref, w2_ref, g2b_ref):
    s = agg_ref[0].astype(jnp.float32) + agg_ref[1].astype(jnp.float32)
    dis = dis_ref[...]
    h1 = jnp.maximum(dis * s + b1_ref[...][None, :], 0.0)
    g2 = dis * jnp.dot(h1, w2_ref[...], preferred_element_type=jnp.float32)
    g2b_ref[...] = g2.astype(jnp.bfloat16)


_dense2 = pl.pallas_call(
    _dense2_body,
    grid=(_GRID,),
    in_specs=[
        pl.BlockSpec((2, _BLK, H), lambda i: (0, i, 0)),
        pl.BlockSpec((_BLK, 1), lambda i: (i, 0)),
        pl.BlockSpec((H,), lambda i: (0,)),
        pl.BlockSpec((H, C), lambda i: (0, 0)),
    ],
    out_specs=pl.BlockSpec((_BLK, C), lambda i: (i, 0)),
    out_shape=jax.ShapeDtypeStruct((N, C), jnp.bfloat16),
)


def _dense3_body(agg_ref, dis_ref, b2_ref, out_ref):
    t = (dis_ref[...] * (agg_ref[0].astype(jnp.float32)
                         + agg_ref[1].astype(jnp.float32))
         + b2_ref[...][None, :])
    m = jnp.max(t, axis=1, keepdims=True)
    lse = m + jnp.log(jnp.sum(jnp.exp(t - m), axis=1, keepdims=True))
    out_ref[...] = t - lse


_dense3 = pl.pallas_call(
    _dense3_body,
    grid=(_GRID,),
    in_specs=[
        pl.BlockSpec((2, _BLK, C), lambda i: (0, i, 0)),
        pl.BlockSpec((_BLK, 1), lambda i: (i, 0)),
        pl.BlockSpec((C,), lambda i: (0,)),
    ],
    out_specs=pl.BlockSpec((_BLK, C), lambda i: (i, 0)),
    out_shape=jax.ShapeDtypeStruct((N, C), jnp.float32),
)


def kernel(x, edge_index, W1, b1, W2, b2):
    padn = E_PAD - E
    rowp = jnp.concatenate([edge_index[0], jnp.full((padn,), N, jnp.int32)])
    colp = jnp.concatenate([edge_index[1], jnp.full((padn,), N, jnp.int32)])
    col3 = colp.reshape(NW, NCHUNKS, CHUNK)

    loop_ids = jnp.arange(N, dtype=jnp.int32)
    padn_a = E_PAD_A - E - N
    rowa = jnp.concatenate([edge_index[0], loop_ids,
                            jnp.full((padn_a,), N, jnp.int32)])
    cola = jnp.concatenate([edge_index[1], loop_ids,
                            jnp.full((padn_a,), N, jnp.int32)])
    row3a = rowa.reshape(NW, NCHUNKS_A, CHUNK)
    col3a = cola.reshape(NW, NCHUNKS_A, CHUNK)

    degp = _sc_degree(col3).reshape(NC, NPAD)
    g1b, dis = _dense1(x, W1, degp)
    g1p = jnp.pad(g1b, ((0, NPAD - N), (0, 0)))
    agg1 = _sc_agg_h(g1p, row3a, col3a)
    g2b = _dense2(agg1, dis, b1, W2)
    g2p = jnp.pad(g2b, ((0, NPAD - N), (0, 0)))
    agg2 = _sc_agg_c(g2p, row3a, col3a)
    return _dense3(agg2, dis, b2)


# speedup vs baseline: 1.0984x; 1.0984x over previous
import functools

import jax
import jax.numpy as jnp
from jax import lax
from jax.experimental import pallas as pl
from jax.experimental.pallas import tpu as pltpu
from jax.experimental.pallas import tpu_sc as plsc

N = 10000
E = 320000
D = 128
H = 128
C = 64

NC = 2
NS = 16
NW = NC * NS

CHUNK = 128
NCHUNKS = 79
EPW = NCHUNKS * CHUNK
E_PAD = NW * EPW
NPAD = 10240
RPT = NPAD // NS
RBLK = RPT // CHUNK
NROW = NPAD // CHUNK

NCHUNKS_A = 81
EPW_A = NCHUNKS_A * CHUNK
E_PAD_A = NW * EPW_A
NBUF = 3

_MESH = plsc.VectorSubcoreMesh(core_axis_name="c", subcore_axis_name="s")


@functools.partial(
    pl.kernel,
    out_type=jax.ShapeDtypeStruct((NC, NROW, CHUNK), jnp.float32),
    mesh=_MESH,
    scratch_types=[
        pltpu.VMEM((NCHUNKS, CHUNK), jnp.int32),
        pltpu.VMEM((NROW, CHUNK), jnp.float32),
        pltpu.VMEM((NROW // NS, CHUNK), jnp.float32),
        pltpu.VMEM((NROW,), jnp.int32),
        pltpu.VMEM_SHARED((NROW, CHUNK), jnp.float32),
    ],
    compiler_params=pltpu.CompilerParams(use_tc_tiling_on_sc=False,
                                         needs_layout_passes=False),
)
def _sc_degree(col_hbm, out_hbm, colbuf, hist, bounce, idbuf, acc):
    cid = lax.axis_index("c")
    sid = lax.axis_index("s")
    wid = sid * NC + cid
    pltpu.sync_copy(col_hbm.at[wid], colbuf)

    zero16 = jnp.zeros((16,), jnp.float32)
    one16 = jnp.ones((16,), jnp.float32)
    for r in range(NROW // NS):
        for j in range(CHUNK // 16):
            bounce[r, pl.ds(j * 16, 16)] = zero16
    for k in range(NROW // 16):
        idbuf[pl.ds(k * 16, 16)] = lax.iota(jnp.int32, 16) + (k * 16)

    def zrow(r, carry):
        for j in range(CHUNK // 16):
            hist[r, pl.ds(j * 16, 16)] = zero16
        return carry

    lax.fori_loop(0, NROW, zrow, 0)
    pltpu.sync_copy(bounce, acc.at[pl.ds(sid * (NROW // NS), NROW // NS)])

    def body(c, carry):
        for j in range(CHUNK // 16):
            idx = colbuf[c, pl.ds(j * 16, 16)]
            plsc.addupdate_scatter(
                hist, [lax.shift_right_logical(idx, 7),
                       lax.bitwise_and(idx, 127)], one16)
        return carry

    lax.fori_loop(0, NCHUNKS, body, 0)
    plsc.subcore_barrier()
    pltpu.sync_copy(hist, acc.at[idbuf], add=True)
    plsc.subcore_barrier()
    pltpu.sync_copy(acc.at[pl.ds(sid * (NROW // NS), NROW // NS)], bounce)
    pltpu.sync_copy(bounce,
                    out_hbm.at[cid, pl.ds(sid * (NROW // NS), NROW // NS)])


def _make_sc_agg(width):
    chunk = CHUNK
    nchunks = NCHUNKS_A
    rblk = RPT // chunk

    @functools.partial(
        pl.kernel,
        out_type=jax.ShapeDtypeStruct((NC, NPAD, width), jnp.bfloat16),
        mesh=_MESH,
        scratch_types=[
            pltpu.VMEM((nchunks, chunk), jnp.int32),
            pltpu.VMEM((nchunks, chunk), jnp.int32),
            pltpu.VMEM((chunk, width), jnp.bfloat16),
            pltpu.VMEM((chunk, width), jnp.bfloat16),
            pltpu.VMEM_SHARED((NPAD, width), jnp.bfloat16),
            pltpu.VMEM_SHARED((NPAD, width), jnp.bfloat16),
            pltpu.SemaphoreType.DMA,
            pltpu.SemaphoreType.DMA,
        ],
        compiler_params=pltpu.CompilerParams(use_tc_tiling_on_sc=False),
    )
    def agg(g_hbm, row_hbm, col_hbm, out_hbm, rowbuf, colbuf,
            buf0, buf1, table, acc, sem_a, sem_b):
        cid = lax.axis_index("c")
        sid = lax.axis_index("s")
        wid = sid * NC + cid
        pltpu.sync_copy(row_hbm.at[wid], rowbuf)
        pltpu.sync_copy(col_hbm.at[wid], colbuf)

        for k in range(rblk):
            pltpu.sync_copy(g_hbm.at[pl.ds(sid * RPT + k * chunk, chunk)],
                            buf0)
            pltpu.sync_copy(buf0, table.at[pl.ds(sid * RPT + k * chunk,
                                                 chunk)])

        zero32 = jnp.zeros((32,), jnp.bfloat16)

        def zrow(r, carry):
            for j in range(width // 32):
                buf0[r, pl.ds(j * 32, 32)] = zero32
            return carry

        lax.fori_loop(0, chunk, zrow, 0)
        for k in range(rblk):
            pltpu.sync_copy(buf0, acc.at[pl.ds(sid * RPT + k * chunk, chunk)])
        plsc.subcore_barrier()

        pltpu.async_copy(table.at[rowbuf.at[0]], buf0, sem_a)

        def body(i, carry):
            c0 = 2 * i
            pltpu.make_async_copy(table.at[rowbuf.at[c0]], buf0, sem_a).wait()
            pltpu.async_copy(table.at[rowbuf.at[c0 + 1]], buf1, sem_b)
            pltpu.sync_copy(buf0, acc.at[colbuf.at[c0]], add=True)
            pltpu.make_async_copy(table.at[rowbuf.at[c0 + 1]], buf1,
                                  sem_b).wait()

            @pl.when(c0 + 2 < nchunks)
            def _():
                pltpu.async_copy(table.at[rowbuf.at[c0 + 2]], buf0, sem_a)

            pltpu.sync_copy(buf1, acc.at[colbuf.at[c0 + 1]], add=True)
            return carry

        lax.fori_loop(0, nchunks // 2, body, 0)
        if nchunks % 2:
            pltpu.make_async_copy(table.at[rowbuf.at[nchunks - 1]], buf0,
                                  sem_a).wait()
            pltpu.sync_copy(buf0, acc.at[colbuf.at[nchunks - 1]], add=True)
        plsc.subcore_barrier()
        for k in range(rblk):
            pltpu.sync_copy(acc.at[pl.ds(sid * RPT + k * chunk, chunk)], buf0)
            pltpu.sync_copy(buf0, out_hbm.at[cid,
                                             pl.ds(sid * RPT + k * chunk,
                                                   chunk)])

    return agg


_sc_agg_h = _make_sc_agg(H)
_sc_agg_c = _make_sc_agg(C)


_BLK = 2048
_GRID = (N + _BLK - 1) // _BLK


def _dense1_body(x_ref, w1_ref, deg_ref, g1b_ref, dis_ref):
    deg = deg_ref[0, :] + deg_ref[1, :] + 1.0
    dis = lax.rsqrt(deg)[:, None]
    g1 = dis * jnp.dot(x_ref[...], w1_ref[...],
                       preferred_element_type=jnp.float32)
    g1b_ref[...] = g1.astype(jnp.bfloat16)
    dis_ref[...] = dis


_dense1 = pl.pallas_call(
    _dense1_body,
    grid=(_GRID,),
    in_specs=[
        pl.BlockSpec((_BLK, D), lambda i: (i, 0)),
        pl.BlockSpec((D, H), lambda i: (0, 0)),
        pl.BlockSpec((2, _BLK), lambda i: (0, i)),
    ],
    out_specs=(pl.BlockSpec((_BLK, H), lambda i: (i, 0)),
               pl.BlockSpec((_BLK, 1), lambda i: (i, 0))),
    out_shape=(jax.ShapeDtypeStruct((N, H), jnp.bfloat16),
               jax.ShapeDtypeStruct((N, 1), jnp.float32)),
)


def _dense2_body(agg_ref, dis_ref, b1_ref, w2_ref, g2b_ref):
    s = agg_ref[0].astype(jnp.float32) + agg_ref[1].astype(jnp.float32)
    dis = dis_ref[...]
    h1 = jnp.maximum(dis * s + b1_ref[...][None, :], 0.0)
    g2 = dis * jnp.dot(h1, w2_ref[...], preferred_element_type=jnp.float32)
    g2b_ref[...] = g2.astype(jnp.bfloat16)


_dense2 = pl.pallas_call(
    _dense2_body,
    grid=(_GRID,),
    in_specs=[
        pl.BlockSpec((2, _BLK, H), lambda i: (0, i, 0)),
        pl.BlockSpec((_BLK, 1), lambda i: (i, 0)),
        pl.BlockSpec((H,), lambda i: (0,)),
        pl.BlockSpec((H, C), lambda i: (0, 0)),
    ],
    out_specs=pl.BlockSpec((_BLK, C), lambda i: (i, 0)),
    out_shape=jax.ShapeDtypeStruct((N, C), jnp.bfloat16),
)


def _dense3_body(agg_ref, dis_ref, b2_ref, out_ref):
    t = (dis_ref[...] * (agg_ref[0].astype(jnp.float32)
                         + agg_ref[1].astype(jnp.float32))
         + b2_ref[...][None, :])
    m = jnp.max(t, axis=1, keepdims=True)
    lse = m + jnp.log(jnp.sum(jnp.exp(t - m), axis=1, keepdims=True))
    out_ref[...] = t - lse


_dense3 = pl.pallas_call(
    _dense3_body,
    grid=(_GRID,),
    in_specs=[
        pl.BlockSpec((2, _BLK, C), lambda i: (0, i, 0)),
        pl.BlockSpec((_BLK, 1), lambda i: (i, 0)),
        pl.BlockSpec((C,), lambda i: (0,)),
    ],
    out_specs=pl.BlockSpec((_BLK, C), lambda i: (i, 0)),
    out_shape=jax.ShapeDtypeStruct((N, C), jnp.float32),
)


def kernel(x, edge_index, W1, b1, W2, b2):
    padn = E_PAD - E
    rowp = jnp.concatenate([edge_index[0], jnp.full((padn,), N, jnp.int32)])
    colp = jnp.concatenate([edge_index[1], jnp.full((padn,), N, jnp.int32)])
    col3 = colp.reshape(NW, NCHUNKS, CHUNK)

    loop_ids = jnp.arange(N, dtype=jnp.int32)
    padn_a = E_PAD_A - E - N
    rowa = jnp.concatenate([edge_index[0], loop_ids,
                            jnp.full((padn_a,), N, jnp.int32)])
    cola = jnp.concatenate([edge_index[1], loop_ids,
                            jnp.full((padn_a,), N, jnp.int32)])
    row3a = rowa.reshape(NW, NCHUNKS_A, CHUNK)
    col3a = cola.reshape(NW, NCHUNKS_A, CHUNK)

    degp = _sc_degree(col3).reshape(NC, NPAD)
    g1b, dis = _dense1(x, W1, degp)
    g1p = jnp.pad(g1b, ((0, NPAD - N), (0, 0)))
    agg1 = _sc_agg_h(g1p, row3a, col3a)
    g2b = _dense2(agg1, dis, b1, W2)
    g2p = jnp.pad(g2b, ((0, NPAD - N), (0, 0)))
    agg2 = _sc_agg_c(g2p, row3a, col3a)
    return _dense3(agg2, dis, b2)


# speedup vs baseline: 1.1042x; 1.0053x over previous
import functools

import jax
import jax.numpy as jnp
from jax import lax
from jax.experimental import pallas as pl
from jax.experimental.pallas import tpu as pltpu
from jax.experimental.pallas import tpu_sc as plsc

N = 10000
E = 320000
D = 128
H = 128
C = 64

NC = 2
NS = 16
NW = NC * NS

CHUNK = 128
NCHUNKS = 79
EPW = NCHUNKS * CHUNK
E_PAD = NW * EPW
NPAD = 10240
RPT = NPAD // NS
RBLK = RPT // CHUNK
NROW = NPAD // CHUNK

NCHUNKS_A = 81
EPW_A = NCHUNKS_A * CHUNK
E_PAD_A = NW * EPW_A
NBUF = 3

_MESH = plsc.VectorSubcoreMesh(core_axis_name="c", subcore_axis_name="s")


@functools.partial(
    pl.kernel,
    out_type=jax.ShapeDtypeStruct((NC, NROW, CHUNK), jnp.float32),
    mesh=_MESH,
    scratch_types=[
        pltpu.VMEM((NCHUNKS, CHUNK), jnp.int32),
        pltpu.VMEM((NROW, CHUNK), jnp.float32),
        pltpu.VMEM((NROW // NS, CHUNK), jnp.float32),
        pltpu.VMEM((NROW,), jnp.int32),
        pltpu.VMEM_SHARED((NROW, CHUNK), jnp.float32),
    ],
    compiler_params=pltpu.CompilerParams(use_tc_tiling_on_sc=False,
                                         needs_layout_passes=False),
)
def _sc_degree(col_hbm, out_hbm, colbuf, hist, bounce, idbuf, acc):
    cid = lax.axis_index("c")
    sid = lax.axis_index("s")
    wid = sid * NC + cid
    pltpu.sync_copy(col_hbm.at[wid], colbuf)

    zero16 = jnp.zeros((16,), jnp.float32)
    one16 = jnp.ones((16,), jnp.float32)
    for r in range(NROW // NS):
        for j in range(CHUNK // 16):
            bounce[r, pl.ds(j * 16, 16)] = zero16
    for k in range(NROW // 16):
        idbuf[pl.ds(k * 16, 16)] = lax.iota(jnp.int32, 16) + (k * 16)

    def zrow(r, carry):
        for j in range(CHUNK // 16):
            hist[r, pl.ds(j * 16, 16)] = zero16
        return carry

    lax.fori_loop(0, NROW, zrow, 0)
    pltpu.sync_copy(bounce, acc.at[pl.ds(sid * (NROW // NS), NROW // NS)])

    def body(c, carry):
        for j in range(CHUNK // 16):
            idx = colbuf[c, pl.ds(j * 16, 16)]
            plsc.addupdate_scatter(
                hist, [lax.shift_right_logical(idx, 7),
                       lax.bitwise_and(idx, 127)], one16)
        return carry

    lax.fori_loop(0, NCHUNKS, body, 0)
    plsc.subcore_barrier()
    pltpu.sync_copy(hist, acc.at[idbuf], add=True)
    plsc.subcore_barrier()
    pltpu.sync_copy(acc.at[pl.ds(sid * (NROW // NS), NROW // NS)], bounce)
    pltpu.sync_copy(bounce,
                    out_hbm.at[cid, pl.ds(sid * (NROW // NS), NROW // NS)])


def _make_sc_agg(width):
    chunk = CHUNK
    nchunks = NCHUNKS_A
    rblk = RPT // chunk

    @functools.partial(
        pl.kernel,
        out_type=jax.ShapeDtypeStruct((NC, NPAD, width), jnp.bfloat16),
        mesh=_MESH,
        scratch_types=[
            pltpu.VMEM((nchunks, chunk), jnp.int32),
            pltpu.VMEM((nchunks, chunk), jnp.int32),
            pltpu.VMEM((chunk, width), jnp.bfloat16),
            pltpu.VMEM((chunk, width), jnp.bfloat16),
            pltpu.VMEM_SHARED((NPAD, width), jnp.bfloat16),
            pltpu.VMEM_SHARED((NPAD, width), jnp.bfloat16),
            pltpu.SemaphoreType.DMA,
            pltpu.SemaphoreType.DMA,
        ],
        compiler_params=pltpu.CompilerParams(use_tc_tiling_on_sc=False),
    )
    def agg(g_hbm, row_hbm, col_hbm, out_hbm, rowbuf, colbuf,
            buf0, buf1, table, acc, sem_a, sem_b):
        cid = lax.axis_index("c")
        sid = lax.axis_index("s")
        wid = sid * NC + cid
        pltpu.sync_copy(row_hbm.at[wid], rowbuf)
        pltpu.sync_copy(col_hbm.at[wid], colbuf)

        for k in range(rblk):
            pltpu.sync_copy(g_hbm.at[pl.ds(sid * RPT + k * chunk, chunk)],
                            buf0)
            pltpu.sync_copy(buf0, table.at[pl.ds(sid * RPT + k * chunk,
                                                 chunk)])

        zero32 = jnp.zeros((32,), jnp.bfloat16)

        def zrow(r, carry):
            for j in range(width // 32):
                buf0[r, pl.ds(j * 32, 32)] = zero32
            return carry

        lax.fori_loop(0, chunk, zrow, 0)
        for k in range(rblk):
            pltpu.sync_copy(buf0, acc.at[pl.ds(sid * RPT + k * chunk, chunk)])
        plsc.subcore_barrier()

        pltpu.async_copy(table.at[rowbuf.at[0]], buf0, sem_a)

        def body(i, carry):
            c0 = 2 * i
            pltpu.make_async_copy(table.at[rowbuf.at[c0]], buf0, sem_a).wait()
            pltpu.async_copy(table.at[rowbuf.at[c0 + 1]], buf1, sem_b)
            pltpu.sync_copy(buf0, acc.at[colbuf.at[c0]], add=True)
            pltpu.make_async_copy(table.at[rowbuf.at[c0 + 1]], buf1,
                                  sem_b).wait()

            @pl.when(c0 + 2 < nchunks)
            def _():
                pltpu.async_copy(table.at[rowbuf.at[c0 + 2]], buf0, sem_a)

            pltpu.sync_copy(buf1, acc.at[colbuf.at[c0 + 1]], add=True)
            return carry

        lax.fori_loop(0, nchunks // 2, body, 0)
        if nchunks % 2:
            pltpu.make_async_copy(table.at[rowbuf.at[nchunks - 1]], buf0,
                                  sem_a).wait()
            pltpu.sync_copy(buf0, acc.at[colbuf.at[nchunks - 1]], add=True)
        plsc.subcore_barrier()
        for k in range(rblk):
            pltpu.sync_copy(acc.at[pl.ds(sid * RPT + k * chunk, chunk)], buf0)
            pltpu.sync_copy(buf0, out_hbm.at[cid,
                                             pl.ds(sid * RPT + k * chunk,
                                                   chunk)])

    return agg


_sc_agg_h = _make_sc_agg(H)
_sc_agg_c = _make_sc_agg(C)


_BLK = 2048
_GRID = (N + _BLK - 1) // _BLK


def _dense1_body(x_ref, w1_ref, deg_ref, g1b_ref, dis_ref):
    deg = deg_ref[0, :N] + deg_ref[1, :N] + 1.0
    dis = lax.rsqrt(deg)[:, None]
    g1 = dis * jnp.dot(x_ref[...], w1_ref[...],
                       preferred_element_type=jnp.float32)
    g1b_ref[...] = g1.astype(jnp.bfloat16)
    dis_ref[...] = dis


_dense1 = pl.pallas_call(
    _dense1_body,
    out_shape=(jax.ShapeDtypeStruct((N, H), jnp.bfloat16),
               jax.ShapeDtypeStruct((N, 1), jnp.float32)),
)


def _dense2_body(agg_ref, dis_ref, b1_ref, w2_ref, g2b_ref):
    s = (agg_ref[0, :N, :].astype(jnp.float32)
         + agg_ref[1, :N, :].astype(jnp.float32))
    dis = dis_ref[...]
    h1 = jnp.maximum(dis * s + b1_ref[...][None, :], 0.0)
    g2 = dis * jnp.dot(h1, w2_ref[...], preferred_element_type=jnp.float32)
    g2b_ref[...] = g2.astype(jnp.bfloat16)


_dense2 = pl.pallas_call(
    _dense2_body,
    out_shape=jax.ShapeDtypeStruct((N, C), jnp.bfloat16),
)


def _dense3_body(agg_ref, dis_ref, b2_ref, out_ref):
    t = (dis_ref[...] * (agg_ref[0, :N, :].astype(jnp.float32)
                         + agg_ref[1, :N, :].astype(jnp.float32))
         + b2_ref[...][None, :])
    m = jnp.max(t, axis=1, keepdims=True)
    lse = m + jnp.log(jnp.sum(jnp.exp(t - m), axis=1, keepdims=True))
    out_ref[...] = t - lse


_dense3 = pl.pallas_call(
    _dense3_body,
    out_shape=jax.ShapeDtypeStruct((N, C), jnp.float32),
)


def kernel(x, edge_index, W1, b1, W2, b2):
    padn = E_PAD - E
    rowp = jnp.concatenate([edge_index[0], jnp.full((padn,), N, jnp.int32)])
    colp = jnp.concatenate([edge_index[1], jnp.full((padn,), N, jnp.int32)])
    col3 = colp.reshape(NW, NCHUNKS, CHUNK)

    loop_ids = jnp.arange(N, dtype=jnp.int32)
    padn_a = E_PAD_A - E - N
    rowa = jnp.concatenate([edge_index[0], loop_ids,
                            jnp.full((padn_a,), N, jnp.int32)])
    cola = jnp.concatenate([edge_index[1], loop_ids,
                            jnp.full((padn_a,), N, jnp.int32)])
    row3a = rowa.reshape(NW, NCHUNKS_A, CHUNK)
    col3a = cola.reshape(NW, NCHUNKS_A, CHUNK)

    degp = _sc_degree(col3).reshape(NC, NPAD)
    g1b, dis = _dense1(x, W1, degp)
    g1p = jnp.pad(g1b, ((0, NPAD - N), (0, 0)))
    agg1 = _sc_agg_h(g1p, row3a, col3a)
    g2b = _dense2(agg1, dis, b1, W2)
    g2p = jnp.pad(g2b, ((0, NPAD - N), (0, 0)))
    agg2 = _sc_agg_c(g2p, row3a, col3a)
    return _dense3(agg2, dis, b2)
